# baseline (speedup 1.0000x reference)
import functools

import jax
import jax.numpy as jnp
import numpy as np
from jax import lax
from jax.experimental import pallas as pl
from jax.experimental.pallas import tpu as pltpu
from jax.experimental.pallas import tpu_sc as plsc

N = 10000
E = 160000
D_IN = 256
D_HID = 256
D_OUT = 512
H = 128
NS = 16
CH = 80
EPT = E // NS
NCH = EPT // CH
NOUT = 10
STRIPE = N // NOUT
ZROWS = 40


@functools.cache
def _make_sc_aggregate(compute_deg: bool):
    mesh = plsc.VectorSubcoreMesh(core_axis_name="c", subcore_axis_name="s")
    out_type = [
        jax.ShapeDtypeStruct((N, H), jnp.float32),
        jax.ShapeDtypeStruct((N, H), jnp.float32),
    ]
    if compute_deg:
        out_type.append(jax.ShapeDtypeStruct((N,), jnp.float32))
    scratch = [
        pltpu.VMEM_SHARED((N, H), jnp.float32),
        pltpu.VMEM((EPT,), jnp.int32),
        pltpu.VMEM((EPT,), jnp.int32),
        pltpu.VMEM((CH, H), jnp.float32),
        pltpu.VMEM((CH, H), jnp.float32),
        pltpu.VMEM((CH,), jnp.int32),
        pltpu.VMEM((CH,), jnp.int32),
        pltpu.VMEM((ZROWS, H), jnp.float32),
        pltpu.SemaphoreType.DMA,
        pltpu.SemaphoreType.DMA,
        pltpu.SemaphoreType.DMA,
        pltpu.SemaphoreType.DMA,
    ]
    if compute_deg:
        scratch += [
            pltpu.VMEM_SHARED((N,), jnp.float32),
            pltpu.VMEM((128,), jnp.float32),
            pltpu.VMEM((STRIPE,), jnp.float32),
            pltpu.SemaphoreType.DMA,
            pltpu.SemaphoreType.DMA,
        ]

    def body(*refs):
        if compute_deg:
            (xil, src, dst, agglo, agghi, deg,
             acc, sidx, didx, rows0, rows1, sidxc0, sidxc1,
             zbuf, gsem0, gsem1, ssem0, ssem1,
             dacc, ones, zdeg, dsem0, dsem1) = refs
        else:
            (xil, src, dst, agglo, agghi,
             acc, sidx, didx, rows0, rows1, sidxc0, sidxc1,
             zbuf, gsem0, gsem1, ssem0, ssem1) = refs
        cid = lax.axis_index("c")
        sid = lax.axis_index("s")
        rows = (rows0, rows1)
        sidxc = (sidxc0, sidxc1)
        gsems = (gsem0, gsem1)
        ssems = (ssem0, ssem1)
        dsems = (dsem0, dsem1) if compute_deg else None
        z16 = jnp.zeros((16,), jnp.float32)

        tbase = pl.multiple_of(sid * EPT, 8)
        pltpu.sync_copy(src.at[pl.ds(tbase, EPT)], sidx)
        pltpu.sync_copy(dst.at[pl.ds(tbase, EPT)], didx)

        def _z_zbuf(k, carry):
            i = k // (H // 16)
            j = k - i * (H // 16)
            zbuf[i, pl.ds(j * 16, 16)] = z16
            return carry
        lax.fori_loop(0, ZROWS * (H // 16), _z_zbuf, 0)

        if compute_deg:
            def _z_zdeg(k, carry):
                zdeg[pl.ds(k * 16, 16)] = z16
                return carry
            lax.fori_loop(0, STRIPE // 16, _z_zdeg, 0)
            zdeg[pl.ds(STRIPE - 16, 16)] = z16
            one16 = jnp.ones((16,), jnp.float32)

            def _fill_ones(k, carry):
                ones[pl.ds(k * 16, 16)] = one16
                return carry
            lax.fori_loop(0, 128 // 16, _fill_ones, 0)

        @pl.when(sid < NOUT)
        def _():
            soff = pl.multiple_of(sid * STRIPE, 8)

            def _zacc(j, carry):
                off = pl.multiple_of(soff + j * ZROWS, 8)
                pltpu.async_copy(zbuf, acc.at[pl.ds(off, ZROWS)], gsem0)
                return carry
            lax.fori_loop(0, STRIPE // ZROWS, _zacc, 0)

            def _zdrain(j, carry):
                pltpu.make_async_copy(zbuf, acc.at[pl.ds(soff, ZROWS)],
                                      gsem0).wait()
                return carry
            lax.fori_loop(0, STRIPE // ZROWS, _zdrain, 0)
            if compute_deg:
                @pl.when(cid == 0)
                def _():
                    pltpu.sync_copy(zdeg, dacc.at[pl.ds(soff, STRIPE)])
        plsc.subcore_barrier()

        def _sl(ref, c):
            return ref.at[pl.ds(pl.multiple_of(c * CH, 8), CH)]

        def _wait_scatter(c, b):
            pltpu.make_async_copy(rows[b], acc.at[_sl(didx, c)],
                                  ssems[b]).wait()

        def _wait_deg(c, b):
            pltpu.make_async_copy(ones.at[pl.ds(0, CH)], dacc.at[_sl(didx, c)],
                                  dsems[b]).wait()

        def _start_gather(c, b):
            @pl.when(c >= 2)
            def _():
                _wait_scatter(c - 2, b)
                if compute_deg:
                    @pl.when(cid == 0)
                    def _():
                        _wait_deg(c - 2, b)
            for k in range(CH // 16):
                sidxc[b][pl.ds(k * 16, 16)] = (
                    sidx[pl.ds(c * CH + k * 16, 16)] * 2 + cid)
            pltpu.async_copy(xil.at[sidxc[b]], rows[b], gsems[b])

        def _wait_gather(c, b):
            pltpu.make_async_copy(xil.at[sidxc[b]], rows[b], gsems[b]).wait()

        def _finish_chunk(c, b):
            _wait_gather(c, b)
            pltpu.async_copy(rows[b], acc.at[_sl(didx, c)], ssems[b], add=True)
            if compute_deg:
                @pl.when(cid == 0)
                def _():
                    pltpu.async_copy(ones.at[pl.ds(0, CH)],
                                     dacc.at[_sl(didx, c)], dsems[b], add=True)

        _start_gather(0, 0)

        def _outer(i, carry):
            c = i * 2
            _start_gather(c + 1, 1)
            _finish_chunk(c, 0)
            _start_gather(c + 2, 0)
            _finish_chunk(c + 1, 1)
            return carry
        lax.fori_loop(0, NCH // 2, _outer, 0)
        _finish_chunk(NCH - 1, 0)
        _wait_scatter(NCH - 1, 0)
        _wait_scatter(NCH - 2, 1)
        if compute_deg:
            @pl.when(cid == 0)
            def _():
                _wait_deg(NCH - 1, 0)
                _wait_deg(NCH - 2, 1)
        plsc.subcore_barrier()

        @pl.when(sid < NOUT)
        def _():
            soff = pl.multiple_of(sid * STRIPE, 8)

            @pl.when(cid == 0)
            def _():
                pltpu.sync_copy(acc.at[pl.ds(soff, STRIPE)],
                                agglo.at[pl.ds(soff, STRIPE)])

            @pl.when(cid == 1)
            def _():
                pltpu.sync_copy(acc.at[pl.ds(soff, STRIPE)],
                                agghi.at[pl.ds(soff, STRIPE)])
            if compute_deg:
                @pl.when(cid == 0)
                def _():
                    pltpu.sync_copy(dacc.at[pl.ds(soff, STRIPE)], zdeg)
                    pltpu.sync_copy(zdeg, deg.at[pl.ds(soff, STRIPE)])

    return pl.kernel(body, out_type=tuple(out_type), mesh=mesh,
                     scratch_types=tuple(scratch))


_R1 = 1000


def _xr_body(x, wr, out):
    out[...] = jnp.dot(x[...], wr[...], preferred_element_type=jnp.float32)


def _l1_body(deg, agl, agh, xr, wl, g, bt, b1, h):
    r = 1.0 / jnp.maximum(deg[...], 1.0)
    aggm = jnp.concatenate([agl[...] * r, agh[...] * r], axis=1)
    acc = jnp.dot(aggm, wl[...], preferred_element_type=jnp.float32) + xr[...]
    gs = g[...] * np.float32(1.0 / np.sqrt(1.0 + 1e-5))
    h[...] = jnp.maximum(acc * gs + (bt[...] + gs * b1[...]), 0.0)


def _hr_body(h, wr, b2, out):
    out[...] = (jnp.dot(h[...], wr[...], preferred_element_type=jnp.float32)
                + b2[...])


def _l2_body(deg, agl, agh, hr, wl, out):
    r = 1.0 / jnp.maximum(deg[...], 1.0)
    aggm = jnp.concatenate([agl[...] * r, agh[...] * r], axis=1)
    out[...] = (jnp.dot(aggm, wl[...], preferred_element_type=jnp.float32)
                + hr[...])


def _row_spec(r, w):
    return pl.BlockSpec((r, w), lambda i: (i, 0))


def _full_spec(shape):
    return pl.BlockSpec(shape, lambda i: (0, 0))


_tc_xr = pl.pallas_call(
    _xr_body,
    grid=(N // _R1,),
    in_specs=[_row_spec(_R1, D_IN), _full_spec((D_IN, D_HID))],
    out_specs=_row_spec(_R1, D_HID),
    out_shape=jax.ShapeDtypeStruct((N, D_HID), jnp.float32),
)

_tc_layer1 = pl.pallas_call(
    _l1_body,
    grid=(N // _R1,),
    in_specs=[
        _row_spec(_R1, 1),
        _row_spec(_R1, H), _row_spec(_R1, H),
        _row_spec(_R1, D_HID),
        _full_spec((D_IN, D_HID)),
        _full_spec((1, D_HID)), _full_spec((1, D_HID)), _full_spec((1, D_HID)),
    ],
    out_specs=_row_spec(_R1, D_HID),
    out_shape=jax.ShapeDtypeStruct((N, D_HID), jnp.float32),
)

_tc_hr = pl.pallas_call(
    _hr_body,
    grid=(N // _R1,),
    in_specs=[
        _row_spec(_R1, D_HID),
        _full_spec((D_HID, D_OUT)), _full_spec((1, D_OUT)),
    ],
    out_specs=_row_spec(_R1, D_OUT),
    out_shape=jax.ShapeDtypeStruct((N, D_OUT), jnp.float32),
)

_tc_layer2 = pl.pallas_call(
    _l2_body,
    grid=(N // _R1,),
    in_specs=[
        _row_spec(_R1, 1),
        _row_spec(_R1, H), _row_spec(_R1, H),
        _row_spec(_R1, D_OUT),
        _full_spec((D_HID, D_OUT)),
    ],
    out_specs=_row_spec(_R1, D_OUT),
    out_shape=jax.ShapeDtypeStruct((N, D_OUT), jnp.float32),
)


def kernel(x, edge_index, Wl1, Wr1, b1, gamma1, beta1, Wl2, Wr2, b2):
    src = edge_index[0]
    dst = edge_index[1]
    xil = x.reshape(2 * N, H)
    agg_lo, agg_hi, deg = _make_sc_aggregate(True)(xil, src, dst)
    xr = _tc_xr(x, Wr1)
    deg2d = deg.reshape(N, 1)
    h = _tc_layer1(deg2d, agg_lo, agg_hi, xr, Wl1,
                   gamma1.reshape(1, -1), beta1.reshape(1, -1),
                   b1.reshape(1, -1))
    a2_lo, a2_hi = _make_sc_aggregate(False)(h.reshape(2 * N, H), src, dst)
    hr = _tc_hr(h, Wr2, b2.reshape(1, -1))
    return _tc_layer2(deg2d, a2_lo, a2_hi, hr, Wl2)


# speedup vs baseline: 1.0654x; 1.0654x over previous
import functools

import jax
import jax.numpy as jnp
import numpy as np
from jax import lax
from jax.experimental import pallas as pl
from jax.experimental.pallas import tpu as pltpu
from jax.experimental.pallas import tpu_sc as plsc

N = 10000
E = 160000
D_IN = 256
D_HID = 256
D_OUT = 512
H = 128
NS = 16
CH = 80
EPT = E // NS
NCH = EPT // CH
NOUT = 10
STRIPE = N // NOUT
ZROWS = 40


@functools.cache
def _make_sc_aggregate(compute_deg: bool):
    mesh = plsc.VectorSubcoreMesh(core_axis_name="c", subcore_axis_name="s")
    out_type = [
        jax.ShapeDtypeStruct((N, H), jnp.float32),
        jax.ShapeDtypeStruct((N, H), jnp.float32),
    ]
    if compute_deg:
        out_type.append(jax.ShapeDtypeStruct((N,), jnp.float32))
    scratch = [
        pltpu.VMEM_SHARED((N, H), jnp.float32),
        pltpu.VMEM((EPT,), jnp.int32),
        pltpu.VMEM((EPT,), jnp.int32),
        pltpu.VMEM((CH, H), jnp.float32),
        pltpu.VMEM((CH, H), jnp.float32),
        pltpu.VMEM((ZROWS, H), jnp.float32),
        pltpu.SemaphoreType.DMA,
        pltpu.SemaphoreType.DMA,
        pltpu.SemaphoreType.DMA,
        pltpu.SemaphoreType.DMA,
    ]
    if compute_deg:
        scratch += [
            pltpu.VMEM_SHARED((N,), jnp.float32),
            pltpu.VMEM((128,), jnp.float32),
            pltpu.VMEM((STRIPE,), jnp.float32),
            pltpu.SemaphoreType.DMA,
            pltpu.SemaphoreType.DMA,
        ]

    def body(*refs):
        if compute_deg:
            (xlo, xhi, src, dst, agglo, agghi, deg,
             acc, sidx, didx, rows0, rows1,
             zbuf, gsem0, gsem1, ssem0, ssem1,
             dacc, ones, zdeg, dsem0, dsem1) = refs
        else:
            (xlo, xhi, src, dst, agglo, agghi,
             acc, sidx, didx, rows0, rows1,
             zbuf, gsem0, gsem1, ssem0, ssem1) = refs
        cid = lax.axis_index("c")
        sid = lax.axis_index("s")
        rows = (rows0, rows1)
        gsems = (gsem0, gsem1)
        ssems = (ssem0, ssem1)
        dsems = (dsem0, dsem1) if compute_deg else None
        z16 = jnp.zeros((16,), jnp.float32)

        tbase = pl.multiple_of(sid * EPT, 8)
        pltpu.sync_copy(src.at[pl.ds(tbase, EPT)], sidx)
        pltpu.sync_copy(dst.at[pl.ds(tbase, EPT)], didx)

        def _z_zbuf(k, carry):
            i = k // (H // 16)
            j = k - i * (H // 16)
            zbuf[i, pl.ds(j * 16, 16)] = z16
            return carry
        lax.fori_loop(0, ZROWS * (H // 16), _z_zbuf, 0)

        if compute_deg:
            def _z_zdeg(k, carry):
                zdeg[pl.ds(k * 16, 16)] = z16
                return carry
            lax.fori_loop(0, STRIPE // 16, _z_zdeg, 0)
            zdeg[pl.ds(STRIPE - 16, 16)] = z16
            one16 = jnp.ones((16,), jnp.float32)

            def _fill_ones(k, carry):
                ones[pl.ds(k * 16, 16)] = one16
                return carry
            lax.fori_loop(0, 128 // 16, _fill_ones, 0)

        @pl.when(sid < NOUT)
        def _():
            soff = pl.multiple_of(sid * STRIPE, 8)

            def _zacc(j, carry):
                off = pl.multiple_of(soff + j * ZROWS, 8)
                pltpu.async_copy(zbuf, acc.at[pl.ds(off, ZROWS)], gsem0)
                return carry
            lax.fori_loop(0, STRIPE // ZROWS, _zacc, 0)

            def _zdrain(j, carry):
                pltpu.make_async_copy(zbuf, acc.at[pl.ds(soff, ZROWS)],
                                      gsem0).wait()
                return carry
            lax.fori_loop(0, STRIPE // ZROWS, _zdrain, 0)
            if compute_deg:
                @pl.when(cid == 0)
                def _():
                    pltpu.sync_copy(zdeg, dacc.at[pl.ds(soff, STRIPE)])
        plsc.subcore_barrier()

        def _sl(ref, c):
            return ref.at[pl.ds(pl.multiple_of(c * CH, 8), CH)]

        def _wait_scatter(c, b):
            pltpu.make_async_copy(rows[b], acc.at[_sl(didx, c)],
                                  ssems[b]).wait()

        def _wait_deg(c, b):
            pltpu.make_async_copy(ones.at[pl.ds(0, CH)], dacc.at[_sl(didx, c)],
                                  dsems[b]).wait()

        def _start_gather(c, b):
            @pl.when(c >= 2)
            def _():
                _wait_scatter(c - 2, b)
                if compute_deg:
                    @pl.when(cid == 0)
                    def _():
                        _wait_deg(c - 2, b)

            @pl.when(cid == 0)
            def _():
                pltpu.async_copy(xlo.at[_sl(sidx, c)], rows[b], gsems[b])

            @pl.when(cid == 1)
            def _():
                pltpu.async_copy(xhi.at[_sl(sidx, c)], rows[b], gsems[b])

        def _wait_gather(c, b):
            pltpu.make_async_copy(xlo.at[_sl(sidx, c)], rows[b],
                                  gsems[b]).wait()

        def _finish_chunk(c, b):
            _wait_gather(c, b)
            pltpu.async_copy(rows[b], acc.at[_sl(didx, c)], ssems[b], add=True)
            if compute_deg:
                @pl.when(cid == 0)
                def _():
                    pltpu.async_copy(ones.at[pl.ds(0, CH)],
                                     dacc.at[_sl(didx, c)], dsems[b], add=True)

        _start_gather(0, 0)

        def _outer(i, carry):
            c = i * 2
            _start_gather(c + 1, 1)
            _finish_chunk(c, 0)
            _start_gather(c + 2, 0)
            _finish_chunk(c + 1, 1)
            return carry
        lax.fori_loop(0, NCH // 2, _outer, 0)
        _finish_chunk(NCH - 1, 0)
        _wait_scatter(NCH - 1, 0)
        _wait_scatter(NCH - 2, 1)
        if compute_deg:
            @pl.when(cid == 0)
            def _():
                _wait_deg(NCH - 1, 0)
                _wait_deg(NCH - 2, 1)
        plsc.subcore_barrier()

        @pl.when(sid < NOUT)
        def _():
            soff = pl.multiple_of(sid * STRIPE, 8)

            @pl.when(cid == 0)
            def _():
                pltpu.sync_copy(acc.at[pl.ds(soff, STRIPE)],
                                agglo.at[pl.ds(soff, STRIPE)])

            @pl.when(cid == 1)
            def _():
                pltpu.sync_copy(acc.at[pl.ds(soff, STRIPE)],
                                agghi.at[pl.ds(soff, STRIPE)])
            if compute_deg:
                @pl.when(cid == 0)
                def _():
                    pltpu.sync_copy(dacc.at[pl.ds(soff, STRIPE)], zdeg)
                    pltpu.sync_copy(zdeg, deg.at[pl.ds(soff, STRIPE)])

    return pl.kernel(body, out_type=tuple(out_type), mesh=mesh,
                     scratch_types=tuple(scratch))


_R1 = 1000
_R2 = 1000


def _l1_body(deg, agl, agh, xl, xh, wl, wr, g, bt, b1, hl, hh):
    r = 1.0 / jnp.maximum(deg[...], 1.0)
    aggm = jnp.concatenate([agl[...] * r, agh[...] * r], axis=1)
    xb = jnp.concatenate([xl[...], xh[...]], axis=1)
    acc = (jnp.dot(aggm, wl[...], preferred_element_type=jnp.float32)
           + jnp.dot(xb, wr[...], preferred_element_type=jnp.float32))
    gs = g[...] * np.float32(1.0 / np.sqrt(1.0 + 1e-5))
    h = jnp.maximum(acc * gs + (bt[...] + gs * b1[...]), 0.0)
    hl[...] = h[:, :H]
    hh[...] = h[:, H:]


def _l2_body(deg, agl, agh, hl, hh, wl, wr, b2, out):
    r = 1.0 / jnp.maximum(deg[...], 1.0)
    aggm = jnp.concatenate([agl[...] * r, agh[...] * r], axis=1)
    hb = jnp.concatenate([hl[...], hh[...]], axis=1)
    out[...] = (jnp.dot(aggm, wl[...], preferred_element_type=jnp.float32)
                + jnp.dot(hb, wr[...], preferred_element_type=jnp.float32)
                + b2[...])


def _row_spec(r, w):
    return pl.BlockSpec((r, w), lambda i: (i, 0))


def _full_spec(shape):
    return pl.BlockSpec(shape, lambda i: (0, 0))


_tc_layer1 = pl.pallas_call(
    _l1_body,
    grid=(N // _R1,),
    in_specs=[
        _row_spec(_R1, 1),
        _row_spec(_R1, H), _row_spec(_R1, H),
        _row_spec(_R1, H), _row_spec(_R1, H),
        _full_spec((D_IN, D_HID)), _full_spec((D_IN, D_HID)),
        _full_spec((1, D_HID)), _full_spec((1, D_HID)), _full_spec((1, D_HID)),
    ],
    out_specs=[_row_spec(_R1, H), _row_spec(_R1, H)],
    out_shape=[jax.ShapeDtypeStruct((N, H), jnp.float32)] * 2,
)

_tc_layer2 = pl.pallas_call(
    _l2_body,
    grid=(N // _R2,),
    in_specs=[
        _row_spec(_R2, 1),
        _row_spec(_R2, H), _row_spec(_R2, H),
        _row_spec(_R2, H), _row_spec(_R2, H),
        _full_spec((D_HID, D_OUT)), _full_spec((D_HID, D_OUT)),
        _full_spec((1, D_OUT)),
    ],
    out_specs=_row_spec(_R2, D_OUT),
    out_shape=jax.ShapeDtypeStruct((N, D_OUT), jnp.float32),
)


def kernel(x, edge_index, Wl1, Wr1, b1, gamma1, beta1, Wl2, Wr2, b2):
    src = edge_index[0]
    dst = edge_index[1]
    xlo = x[:, :H]
    xhi = x[:, H:]
    agg_lo, agg_hi, deg = _make_sc_aggregate(True)(xlo, xhi, src, dst)
    deg2d = deg.reshape(N, 1)
    h_lo, h_hi = _tc_layer1(deg2d, agg_lo, agg_hi, xlo, xhi, Wl1, Wr1,
                            gamma1.reshape(1, -1), beta1.reshape(1, -1),
                            b1.reshape(1, -1))
    a2_lo, a2_hi = _make_sc_aggregate(False)(h_lo, h_hi, src, dst)
    return _tc_layer2(deg2d, a2_lo, a2_hi, h_lo, h_hi, Wl2, Wr2,
                      b2.reshape(1, -1))
